# baseline (speedup 1.0000x reference)
import functools

import jax
import jax.numpy as jnp
from jax import lax
from jax.experimental import pallas as pl
from jax.experimental.pallas import tpu as pltpu
from jax.experimental.pallas import tpu_sc as plsc

N, E, D, H, G = 10000, 320000, 128, 128, 512
NSEG = 64
NC, NS = 2, 16
CHUNK = 128
NCH0 = 112
NCH1 = 48
NCHD = 80
GRP = 16
CH0_TOT = NS * NCH0
CH_TOT = NS * (NCH0 + NCH1)
E_PAD = CH_TOT * CHUNK
ROWS_PT = 640
ACC_N = NS * ROWS_PT
BLK = 1000
NB = N // BLK


def _sc_deg_body(dst_hbm, out_hbm, idx_v, ones_v, zeros_v, acc):
    c = lax.axis_index("c")
    s = lax.axis_index("s")
    for k in range(CHUNK // 16):
        ones_v[pl.ds(k * 16, 16)] = jnp.ones((16,), jnp.float32)
    for k in range(ROWS_PT // 16):
        zeros_v[pl.ds(k * 16, 16)] = jnp.zeros((16,), jnp.float32)
    pltpu.sync_copy(zeros_v, acc.at[pl.ds(s * ROWS_PT, ROWS_PT)])
    base = pl.multiple_of((c * NS + s) * NCHD, 8)
    pltpu.sync_copy(dst_hbm.at[pl.ds(base, NCHD)], idx_v)
    plsc.subcore_barrier()

    def body(j, carry):
        pltpu.sync_copy(ones_v, acc.at[idx_v.at[j]], add=True)
        return carry

    lax.fori_loop(0, NCHD, body, 0)
    plsc.subcore_barrier()

    @pl.when(s == 0)
    def _():
        pltpu.sync_copy(acc, out_hbm.at[c])


def _sc_deg(dst_pad):
    mesh = plsc.VectorSubcoreMesh(core_axis_name="c", subcore_axis_name="s", num_cores=NC, num_subcores=NS)
    f = pl.kernel(
        _sc_deg_body,
        out_type=jax.ShapeDtypeStruct((NC, ACC_N), jnp.float32),
        mesh=mesh,
        scratch_types=[
            pltpu.VMEM((NCHD, CHUNK), jnp.int32),
            pltpu.VMEM((CHUNK,), jnp.float32),
            pltpu.VMEM((ROWS_PT,), jnp.float32),
            pltpu.VMEM_SHARED((ACC_N,), jnp.float32),
        ],
    )
    return f(dst_pad)


def _sc_mp_body(src_hbm, dst_hbm, u_hbm, out_hbm, srcv, dstv,
                rows0, rows1, sem0, sem1, acc):
    c = lax.axis_index("c")
    s = lax.axis_index("s")

    def zrow(i, carry):
        for k in range(H // 16):
            rows0[i, pl.ds(k * 16, 16)] = jnp.zeros((16,), jnp.float32)
        return carry

    lax.fori_loop(0, CHUNK, zrow, 0)
    for k in range(ROWS_PT // CHUNK):
        pltpu.sync_copy(rows0, acc.at[pl.ds(s * ROWS_PT + k * CHUNK, CHUNK)])

    plsc.subcore_barrier()

    base0 = s * NCH0
    base1 = CH0_TOT + s * NCH1

    @pl.when(c == 0)
    def _():
        def group(g, carry):
            gofs = pl.multiple_of(base0 + g * GRP, 8)
            pltpu.sync_copy(src_hbm.at[pl.ds(gofs, GRP)], srcv)
            pltpu.sync_copy(dst_hbm.at[pl.ds(gofs, GRP)], dstv)
            pltpu.async_copy(u_hbm.at[srcv.at[0]], rows0, sem0)

            def body(jj, carry2):
                j = 2 * jj
                pltpu.make_async_copy(u_hbm.at[srcv.at[j]], rows0, sem0).wait()
                pltpu.async_copy(u_hbm.at[srcv.at[j + 1]], rows1, sem1)
                pltpu.sync_copy(rows0, acc.at[dstv.at[j]], add=True)
                pltpu.make_async_copy(
                    u_hbm.at[srcv.at[j + 1]], rows1, sem1).wait()

                @pl.when(jj < GRP // 2 - 1)
                def _():
                    pltpu.async_copy(u_hbm.at[srcv.at[j + 2]], rows0, sem0)

                pltpu.sync_copy(rows1, acc.at[dstv.at[j + 1]], add=True)
                return carry2

            lax.fori_loop(0, GRP // 2, body, 0)
            return carry

        lax.fori_loop(0, NCH0 // GRP, group, 0)

    @pl.when(c == 1)
    def _():
        def group(g, carry):
            gofs = pl.multiple_of(base1 + g * GRP, 8)
            pltpu.sync_copy(src_hbm.at[pl.ds(gofs, GRP)], srcv)
            pltpu.sync_copy(dst_hbm.at[pl.ds(gofs, GRP)], dstv)

            def body(j, carry2):
                pltpu.async_copy(u_hbm.at[srcv.at[j]], rows0, sem0).wait()
                pltpu.sync_copy(rows0, acc.at[dstv.at[j]], add=True)
                return carry2

            lax.fori_loop(0, GRP, body, 0)
            return carry

        lax.fori_loop(0, NCH1 // GRP, group, 0)

    plsc.subcore_barrier()
    pltpu.sync_copy(acc.at[pl.ds(s * ROWS_PT, ROWS_PT)],
                    out_hbm.at[c, pl.ds(s * ROWS_PT, ROWS_PT)])


def _sc_mp(src_pad, dst_pad, u):
    mesh = plsc.VectorSubcoreMesh(core_axis_name="c", subcore_axis_name="s", num_cores=NC, num_subcores=NS)
    f = pl.kernel(
        _sc_mp_body,
        out_type=jax.ShapeDtypeStruct((NC, ACC_N, H), jnp.float32),
        mesh=mesh,
        scratch_types=[
            pltpu.VMEM((GRP, CHUNK), jnp.int32),
            pltpu.VMEM((GRP, CHUNK), jnp.int32),
            pltpu.VMEM((CHUNK, H), jnp.float32),
            pltpu.VMEM((CHUNK, H), jnp.float32),
            pltpu.SemaphoreType.DMA,
            pltpu.SemaphoreType.DMA,
            pltpu.VMEM_SHARED((ACC_N, H), jnp.float32),
        ],
    )
    return f(src_pad, dst_pad, u)


def _tmin_body(x_ref, out_ref, m_ref):
    i = pl.program_id(0)

    @pl.when(i == 0)
    def _():
        m_ref[0] = jnp.int32(2147483647)

    t = x_ref[...].astype(jnp.int32)
    m_ref[0] = jnp.minimum(m_ref[0], jnp.min(t[:, 5:6]))

    @pl.when(i == NB - 1)
    def _():
        out_ref[0, 0] = m_ref[0]


def _tc_tmin(x):
    return pl.pallas_call(
        _tmin_body,
        grid=(NB,),
        in_specs=[pl.BlockSpec((BLK, D), lambda i: (i, 0))],
        out_specs=pl.BlockSpec(memory_space=pltpu.SMEM),
        out_shape=jax.ShapeDtypeStruct((1, 1), jnp.int32),
        scratch_shapes=[pltpu.SMEM((1,), jnp.int32)],
    )(x)


def _seg_body(tmin_ref, x_ref, sums_ref, cnts_ref):
    i = pl.program_id(0)

    @pl.when(i == 0)
    def _():
        sums_ref[...] = jnp.zeros_like(sums_ref)
        cnts_ref[...] = jnp.zeros_like(cnts_ref)

    xb = x_ref[...]
    ids = xb[:, 5:6].astype(jnp.int32) - tmin_ref[0, 0]
    iot = lax.broadcasted_iota(jnp.int32, (BLK, NSEG), 1)
    oh = jnp.where((iot == ids) & (ids < NSEG), 1.0, 0.0)
    dn = (((0,), (0,)), ((), ()))
    sums_ref[...] += lax.dot_general(oh, xb, dn,
                                     preferred_element_type=jnp.float32)
    cnts_ref[...] += lax.dot_general(oh, jnp.ones((BLK, 1), jnp.float32), dn,
                                     preferred_element_type=jnp.float32)


def _tc_seg(x, tmin):
    return pl.pallas_call(
        _seg_body,
        grid=(NB,),
        in_specs=[
            pl.BlockSpec(memory_space=pltpu.SMEM),
            pl.BlockSpec((BLK, D), lambda i: (i, 0)),
        ],
        out_specs=[
            pl.BlockSpec((NSEG, D), lambda i: (0, 0)),
            pl.BlockSpec((NSEG, 1), lambda i: (0, 0)),
        ],
        out_shape=[
            jax.ShapeDtypeStruct((NSEG, D), jnp.float32),
            jax.ShapeDtypeStruct((NSEG, 1), jnp.float32),
        ],
    )(tmin, x)


def _attn_body(sums_ref, cnts_ref, wa1_ref, ba1_ref, wa2_ref, out_ref):
    cnts = cnts_ref[...]
    means = sums_ref[...] / jnp.maximum(cnts, 1.0)
    hmid = jnp.tanh(jnp.dot(means, wa1_ref[...],
                            preferred_element_type=jnp.float32) + ba1_ref[...])
    scores = jnp.dot(hmid, wa2_ref[...],
                     preferred_element_type=jnp.float32)
    scores = jnp.where(cnts > 0.0, scores, -1e30)
    mx = jnp.max(scores, axis=0, keepdims=True)
    e = jnp.exp(scores - mx)
    out_ref[...] = e / jnp.sum(e, axis=0, keepdims=True)


def _tc_attn(sums, cnts, W_a1, b_a1, W_a2):
    return pl.pallas_call(
        _attn_body,
        out_shape=jax.ShapeDtypeStruct((NSEG, 1), jnp.float32),
    )(sums, cnts, W_a1, b_a1.reshape(1, 64), W_a2)


def _k0_body(tmin_ref, x_ref, degp_ref, subst_ref, w0a_ref, w0b_ref,
             u_ref, dinv_ref):
    xb = x_ref[...]
    deg = degp_ref[:, 0:1] + degp_ref[:, 1:2] + 1.0
    dinv = lax.rsqrt(deg)
    ids = xb[:, 5:6].astype(jnp.int32) - tmin_ref[0, 0]
    ids = jnp.minimum(ids, NSEG - 1)
    iot = lax.broadcasted_iota(jnp.int32, (BLK, NSEG), 1)
    oh = jnp.where(iot == ids, 1.0, 0.0)
    attn = jnp.dot(oh, subst_ref[...],
                   preferred_element_type=jnp.float32)
    u = jnp.dot(xb, w0a_ref[...], preferred_element_type=jnp.float32)
    u = dinv * (u + attn * w0b_ref[...])
    u_ref[...] = u
    dinv_ref[...] = dinv


def _tc_k0(x, degpT, subst, W0a, w0b, tmin):
    return pl.pallas_call(
        _k0_body,
        grid=(NB,),
        in_specs=[
            pl.BlockSpec(memory_space=pltpu.SMEM),
            pl.BlockSpec((BLK, D), lambda i: (i, 0)),
            pl.BlockSpec((BLK, NC), lambda i: (i, 0)),
            pl.BlockSpec((NSEG, 1), lambda i: (0, 0)),
            pl.BlockSpec((D, H), lambda i: (0, 0)),
            pl.BlockSpec((1, H), lambda i: (0, 0)),
        ],
        out_specs=[
            pl.BlockSpec((BLK, H), lambda i: (i, 0)),
            pl.BlockSpec((BLK, 1), lambda i: (i, 0)),
        ],
        out_shape=[
            jax.ShapeDtypeStruct((N, H), jnp.float32),
            jax.ShapeDtypeStruct((N, 1), jnp.float32),
        ],
    )(tmin, x, degpT, subst, W0a, w0b)


def _comb_body(p_ref, u_ref, dinv_ref, b_ref, agg_ref, st_ref):
    i = pl.program_id(0)
    agg = dinv_ref[...] * (p_ref[0] + p_ref[1] + u_ref[...]) + b_ref[...]
    agg_ref[...] = agg

    @pl.when(i == 0)
    def _():
        st_ref[...] = jnp.zeros_like(st_ref)

    st_ref[...] += jnp.concatenate(
        [jnp.sum(agg, axis=0, keepdims=True),
         jnp.sum(agg * agg, axis=0, keepdims=True)], axis=0)


def _tc_comb(p, u, dinv, b):
    return pl.pallas_call(
        _comb_body,
        grid=(NB,),
        in_specs=[
            pl.BlockSpec((NC, BLK, H), lambda i: (0, i, 0)),
            pl.BlockSpec((BLK, H), lambda i: (i, 0)),
            pl.BlockSpec((BLK, 1), lambda i: (i, 0)),
            pl.BlockSpec((1, H), lambda i: (0, 0)),
        ],
        out_specs=[
            pl.BlockSpec((BLK, H), lambda i: (i, 0)),
            pl.BlockSpec((2, H), lambda i: (0, 0)),
        ],
        out_shape=[
            jax.ShapeDtypeStruct((N, H), jnp.float32),
            jax.ShapeDtypeStruct((2, H), jnp.float32),
        ],
    )(p, u, dinv, b.reshape(1, H))


def _norm_body(agg_ref, st_ref, g_ref, bt_ref, wn_ref, dinv_ref, u_ref):
    st = st_ref[...]
    m = st[0:1] * (1.0 / N)
    var = st[1:2] * (1.0 / N) - m * m
    rstd = lax.rsqrt(var + 1e-5)
    a = (agg_ref[...] - m) * (rstd * g_ref[...]) + bt_ref[...]
    a = jnp.where(a > 0.0, a, jnp.exp(a) - 1.0)
    u_ref[...] = dinv_ref[...] * jnp.dot(a, wn_ref[...],
                                         preferred_element_type=jnp.float32)


def _tc_norm(agg, st, g, bt, wn, dinv):
    return pl.pallas_call(
        _norm_body,
        grid=(NB,),
        in_specs=[
            pl.BlockSpec((BLK, H), lambda i: (i, 0)),
            pl.BlockSpec((2, H), lambda i: (0, 0)),
            pl.BlockSpec((1, H), lambda i: (0, 0)),
            pl.BlockSpec((1, H), lambda i: (0, 0)),
            pl.BlockSpec((H, H), lambda i: (0, 0)),
            pl.BlockSpec((BLK, 1), lambda i: (i, 0)),
        ],
        out_specs=pl.BlockSpec((BLK, H), lambda i: (i, 0)),
        out_shape=jax.ShapeDtypeStruct((N, H), jnp.float32),
    )(agg, st, g.reshape(1, H), bt.reshape(1, H), wn, dinv)


def _pool_body(agg_ref, st_ref, g_ref, bt_ref, batch_ref, ps_ref, pc_ref):
    i = pl.program_id(0)
    st = st_ref[...]
    m = st[0:1] * (1.0 / N)
    var = st[1:2] * (1.0 / N) - m * m
    rstd = lax.rsqrt(var + 1e-5)
    a = (agg_ref[...] - m) * (rstd * g_ref[...]) + bt_ref[...]
    a = jnp.maximum(a, 0.0)

    @pl.when(i == 0)
    def _():
        ps_ref[...] = jnp.zeros_like(ps_ref)
        pc_ref[...] = jnp.zeros_like(pc_ref)

    ids = batch_ref[...]
    iot = lax.broadcasted_iota(jnp.int32, (BLK, G), 1)
    oh = jnp.where(iot == ids, 1.0, 0.0)
    dn = (((0,), (0,)), ((), ()))
    ps_ref[...] += lax.dot_general(oh, a, dn,
                                   preferred_element_type=jnp.float32)
    pc_ref[...] += lax.dot_general(oh, jnp.ones((BLK, 1), jnp.float32), dn,
                                   preferred_element_type=jnp.float32)


def _tc_pool(agg, st, g, bt, batch2d):
    return pl.pallas_call(
        _pool_body,
        grid=(NB,),
        in_specs=[
            pl.BlockSpec((BLK, H), lambda i: (i, 0)),
            pl.BlockSpec((2, H), lambda i: (0, 0)),
            pl.BlockSpec((1, H), lambda i: (0, 0)),
            pl.BlockSpec((1, H), lambda i: (0, 0)),
            pl.BlockSpec((BLK, 1), lambda i: (i, 0)),
        ],
        out_specs=[
            pl.BlockSpec((G, H), lambda i: (0, 0)),
            pl.BlockSpec((G, 1), lambda i: (0, 0)),
        ],
        out_shape=[
            jax.ShapeDtypeStruct((G, H), jnp.float32),
            jax.ShapeDtypeStruct((G, 1), jnp.float32),
        ],
    )(agg, st, g.reshape(1, H), bt.reshape(1, H), batch2d)


def _final_body(ps_ref, pc_ref, wp1_ref, bp1_ref, wp2_ref, bp2_ref, out_ref):
    pooled = ps_ref[...] / jnp.maximum(pc_ref[...], 1.0)
    h1 = jnp.dot(pooled, wp1_ref[...],
                 preferred_element_type=jnp.float32) + bp1_ref[...]
    h1 = jnp.where(h1 > 0.0, h1, jnp.exp(h1) - 1.0)
    out_ref[...] = jnp.dot(h1, wp2_ref[...],
                           preferred_element_type=jnp.float32) + bp2_ref[...]


def _tc_final(ps, pc, Wp1, bp1, Wp2, bp2):
    return pl.pallas_call(
        _final_body,
        out_shape=jax.ShapeDtypeStruct((G, 1), jnp.float32),
    )(ps, pc, Wp1, bp1.reshape(1, H // 2), Wp2, bp2.reshape(1, 1))


def kernel(x, edge_index, batch, W_a1, b_a1, W_a2,
           Wg0, bg0, gamma0, beta0, Wg1, bg1, gamma1, beta1,
           Wg2, bg2, gamma2, beta2, Wg3, bg3, gamma3, beta3,
           Wp1, bp1, Wp2, bp2):
    pad = E_PAD - E
    src_pad = jnp.concatenate(
        [edge_index[0], jnp.zeros((pad,), jnp.int32)]).reshape(CH_TOT, CHUNK)
    dst_pad = jnp.concatenate(
        [edge_index[1], jnp.full((pad,), N, jnp.int32)]).reshape(CH_TOT, CHUNK)
    batch2d = batch.reshape(N, 1)

    tmin = _tc_tmin(x)
    sums, cnts = _tc_seg(x, tmin)
    subst = _tc_attn(sums, cnts, W_a1, b_a1, W_a2)

    degp = _sc_deg(dst_pad)
    degpT = degp.T

    W0a = Wg0[:D]
    w0b = Wg0[D:D + 1]
    u, dinv = _tc_k0(x, degpT, subst, W0a, w0b, tmin)

    layers = [(bg0, gamma0, beta0, Wg1), (bg1, gamma1, beta1, Wg2),
              (bg2, gamma2, beta2, Wg3)]
    for b, g, bt, wn in layers:
        p = _sc_mp(src_pad, dst_pad, u)
        agg, st = _tc_comb(p, u, dinv, b)
        u = _tc_norm(agg, st, g, bt, wn, dinv)

    p = _sc_mp(src_pad, dst_pad, u)
    agg, st = _tc_comb(p, u, dinv, bg3)
    ps, pc = _tc_pool(agg, st, gamma3, beta3, batch2d)
    return _tc_final(ps, pc, Wp1, bp1, Wp2, bp2)


# speedup vs baseline: 3.2830x; 3.2830x over previous
import functools

import jax
import jax.numpy as jnp
from jax import lax
from jax.experimental import pallas as pl
from jax.experimental.pallas import tpu as pltpu
from jax.experimental.pallas import tpu_sc as plsc

N, E, D, H, G = 10000, 320000, 128, 128, 512
NSEG = 64
NC, NS = 2, 16
CHUNK = 128
NCH = 80
NCHD = 80
GRP = 16
CH_TOT = NC * NS * NCH
E_PAD = CH_TOT * CHUNK
ROWS_PT = 640
ACC_N = NS * ROWS_PT
BLK = 1000
NB = N // BLK


def _sc_deg_body(dst_hbm, out_hbm, idx_v, ones_v, zeros_v, acc):
    c = lax.axis_index("c")
    s = lax.axis_index("s")
    for k in range(CHUNK // 16):
        ones_v[pl.ds(k * 16, 16)] = jnp.ones((16,), jnp.float32)
    for k in range(ROWS_PT // 16):
        zeros_v[pl.ds(k * 16, 16)] = jnp.zeros((16,), jnp.float32)
    pltpu.sync_copy(zeros_v, acc.at[pl.ds(s * ROWS_PT, ROWS_PT)])
    base = pl.multiple_of((c * NS + s) * NCHD, 8)
    pltpu.sync_copy(dst_hbm.at[pl.ds(base, NCHD)], idx_v)
    plsc.subcore_barrier()

    def body(j, carry):
        pltpu.sync_copy(ones_v, acc.at[idx_v.at[j]], add=True)
        return carry

    lax.fori_loop(0, NCHD, body, 0)
    plsc.subcore_barrier()

    @pl.when(s == 0)
    def _():
        pltpu.sync_copy(acc, out_hbm.at[c])


def _sc_deg(dst_pad):
    mesh = plsc.VectorSubcoreMesh(core_axis_name="c", subcore_axis_name="s", num_cores=NC, num_subcores=NS)
    f = pl.kernel(
        _sc_deg_body,
        out_type=jax.ShapeDtypeStruct((NC, ACC_N), jnp.float32),
        mesh=mesh,
        scratch_types=[
            pltpu.VMEM((NCHD, CHUNK), jnp.int32),
            pltpu.VMEM((CHUNK,), jnp.float32),
            pltpu.VMEM((ROWS_PT,), jnp.float32),
            pltpu.VMEM_SHARED((ACC_N,), jnp.float32),
        ],
    )
    return f(dst_pad)


def _sc_mp_body(src_hbm, dst_hbm, u_hbm, out_hbm, srcv, dstv,
                rows0, rows1, sem0, sem1, acc):
    c = lax.axis_index("c")
    s = lax.axis_index("s")

    def zrow(i, carry):
        for k in range(H // 16):
            rows0[i, pl.ds(k * 16, 16)] = jnp.zeros((16,), jnp.float32)
        return carry

    lax.fori_loop(0, CHUNK, zrow, 0)
    for k in range(ROWS_PT // CHUNK):
        pltpu.sync_copy(rows0, acc.at[pl.ds(s * ROWS_PT + k * CHUNK, CHUNK)])

    plsc.subcore_barrier()

    base = (c * NS + s) * NCH

    def group(g, carry):
        gofs = pl.multiple_of(base + g * GRP, 8)
        pltpu.sync_copy(src_hbm.at[pl.ds(gofs, GRP)], srcv)
        pltpu.sync_copy(dst_hbm.at[pl.ds(gofs, GRP)], dstv)
        pltpu.async_copy(u_hbm.at[srcv.at[0]], rows0, sem0)

        def body(jj, carry2):
            j = 2 * jj
            pltpu.make_async_copy(u_hbm.at[srcv.at[j]], rows0, sem0).wait()
            pltpu.async_copy(u_hbm.at[srcv.at[j + 1]], rows1, sem1)
            pltpu.sync_copy(rows0, acc.at[dstv.at[j]], add=True)
            pltpu.make_async_copy(u_hbm.at[srcv.at[j + 1]], rows1, sem1).wait()

            @pl.when(jj < GRP // 2 - 1)
            def _():
                pltpu.async_copy(u_hbm.at[srcv.at[j + 2]], rows0, sem0)

            pltpu.sync_copy(rows1, acc.at[dstv.at[j + 1]], add=True)
            return carry2

        lax.fori_loop(0, GRP // 2, body, 0)
        return carry

    lax.fori_loop(0, NCH // GRP, group, 0)
    plsc.subcore_barrier()
    pltpu.sync_copy(acc.at[pl.ds(s * ROWS_PT, ROWS_PT)],
                    out_hbm.at[c, pl.ds(s * ROWS_PT, ROWS_PT)])


def _sc_mp(src_pad, dst_pad, u):
    mesh = plsc.VectorSubcoreMesh(core_axis_name="c", subcore_axis_name="s", num_cores=NC, num_subcores=NS)
    f = pl.kernel(
        _sc_mp_body,
        out_type=jax.ShapeDtypeStruct((NC, ACC_N, H), jnp.float32),
        mesh=mesh,
        scratch_types=[
            pltpu.VMEM((GRP, CHUNK), jnp.int32),
            pltpu.VMEM((GRP, CHUNK), jnp.int32),
            pltpu.VMEM((CHUNK, H), jnp.float32),
            pltpu.VMEM((CHUNK, H), jnp.float32),
            pltpu.SemaphoreType.DMA,
            pltpu.SemaphoreType.DMA,
            pltpu.VMEM_SHARED((ACC_N, H), jnp.float32),
        ],
    )
    return f(src_pad, dst_pad, u)


def _tmin_body(x_ref, out_ref, m_ref):
    i = pl.program_id(0)

    @pl.when(i == 0)
    def _():
        m_ref[0] = jnp.int32(2147483647)

    t = x_ref[...].astype(jnp.int32)
    m_ref[0] = jnp.minimum(m_ref[0], jnp.min(t[:, 5:6]))

    @pl.when(i == NB - 1)
    def _():
        out_ref[0, 0] = m_ref[0]


def _tc_tmin(x):
    return pl.pallas_call(
        _tmin_body,
        grid=(NB,),
        in_specs=[pl.BlockSpec((BLK, D), lambda i: (i, 0))],
        out_specs=pl.BlockSpec(memory_space=pltpu.SMEM),
        out_shape=jax.ShapeDtypeStruct((1, 1), jnp.int32),
        scratch_shapes=[pltpu.SMEM((1,), jnp.int32)],
    )(x)


def _seg_body(tmin_ref, x_ref, sums_ref, cnts_ref):
    i = pl.program_id(0)

    @pl.when(i == 0)
    def _():
        sums_ref[...] = jnp.zeros_like(sums_ref)
        cnts_ref[...] = jnp.zeros_like(cnts_ref)

    xb = x_ref[...]
    ids = xb[:, 5:6].astype(jnp.int32) - tmin_ref[0, 0]
    iot = lax.broadcasted_iota(jnp.int32, (BLK, NSEG), 1)
    oh = jnp.where((iot == ids) & (ids < NSEG), 1.0, 0.0)
    dn = (((0,), (0,)), ((), ()))
    sums_ref[...] += lax.dot_general(oh, xb, dn,
                                     preferred_element_type=jnp.float32)
    cnts_ref[...] += lax.dot_general(oh, jnp.ones((BLK, 1), jnp.float32), dn,
                                     preferred_element_type=jnp.float32)


def _tc_seg(x, tmin):
    return pl.pallas_call(
        _seg_body,
        grid=(NB,),
        in_specs=[
            pl.BlockSpec(memory_space=pltpu.SMEM),
            pl.BlockSpec((BLK, D), lambda i: (i, 0)),
        ],
        out_specs=[
            pl.BlockSpec((NSEG, D), lambda i: (0, 0)),
            pl.BlockSpec((NSEG, 1), lambda i: (0, 0)),
        ],
        out_shape=[
            jax.ShapeDtypeStruct((NSEG, D), jnp.float32),
            jax.ShapeDtypeStruct((NSEG, 1), jnp.float32),
        ],
    )(tmin, x)


def _attn_body(sums_ref, cnts_ref, wa1_ref, ba1_ref, wa2_ref, out_ref):
    cnts = cnts_ref[...]
    means = sums_ref[...] / jnp.maximum(cnts, 1.0)
    hmid = jnp.tanh(jnp.dot(means, wa1_ref[...],
                            preferred_element_type=jnp.float32) + ba1_ref[...])
    scores = jnp.dot(hmid, wa2_ref[...],
                     preferred_element_type=jnp.float32)
    scores = jnp.where(cnts > 0.0, scores, -1e30)
    mx = jnp.max(scores, axis=0, keepdims=True)
    e = jnp.exp(scores - mx)
    out_ref[...] = e / jnp.sum(e, axis=0, keepdims=True)


def _tc_attn(sums, cnts, W_a1, b_a1, W_a2):
    return pl.pallas_call(
        _attn_body,
        out_shape=jax.ShapeDtypeStruct((NSEG, 1), jnp.float32),
    )(sums, cnts, W_a1, b_a1.reshape(1, 64), W_a2)


def _k0_body(tmin_ref, x_ref, degp_ref, subst_ref, w0a_ref, w0b_ref,
             u_ref, dinv_ref):
    xb = x_ref[...]
    deg = degp_ref[:, 0:1] + degp_ref[:, 1:2] + 1.0
    dinv = lax.rsqrt(deg)
    ids = xb[:, 5:6].astype(jnp.int32) - tmin_ref[0, 0]
    ids = jnp.minimum(ids, NSEG - 1)
    iot = lax.broadcasted_iota(jnp.int32, (BLK, NSEG), 1)
    oh = jnp.where(iot == ids, 1.0, 0.0)
    attn = jnp.dot(oh, subst_ref[...],
                   preferred_element_type=jnp.float32)
    u = jnp.dot(xb, w0a_ref[...], preferred_element_type=jnp.float32)
    u = dinv * (u + attn * w0b_ref[...])
    u_ref[...] = u
    dinv_ref[...] = dinv


def _tc_k0(x, degpT, subst, W0a, w0b, tmin):
    return pl.pallas_call(
        _k0_body,
        grid=(NB,),
        in_specs=[
            pl.BlockSpec(memory_space=pltpu.SMEM),
            pl.BlockSpec((BLK, D), lambda i: (i, 0)),
            pl.BlockSpec((BLK, NC), lambda i: (i, 0)),
            pl.BlockSpec((NSEG, 1), lambda i: (0, 0)),
            pl.BlockSpec((D, H), lambda i: (0, 0)),
            pl.BlockSpec((1, H), lambda i: (0, 0)),
        ],
        out_specs=[
            pl.BlockSpec((BLK, H), lambda i: (i, 0)),
            pl.BlockSpec((BLK, 1), lambda i: (i, 0)),
        ],
        out_shape=[
            jax.ShapeDtypeStruct((N, H), jnp.float32),
            jax.ShapeDtypeStruct((N, 1), jnp.float32),
        ],
    )(tmin, x, degpT, subst, W0a, w0b)


def _comb_body(p_ref, u_ref, dinv_ref, b_ref, agg_ref, st_ref):
    i = pl.program_id(0)
    agg = dinv_ref[...] * (p_ref[0] + p_ref[1] + u_ref[...]) + b_ref[...]
    agg_ref[...] = agg

    @pl.when(i == 0)
    def _():
        st_ref[...] = jnp.zeros_like(st_ref)

    st_ref[...] += jnp.concatenate(
        [jnp.sum(agg, axis=0, keepdims=True),
         jnp.sum(agg * agg, axis=0, keepdims=True)], axis=0)


def _tc_comb(p, u, dinv, b):
    return pl.pallas_call(
        _comb_body,
        grid=(NB,),
        in_specs=[
            pl.BlockSpec((NC, BLK, H), lambda i: (0, i, 0)),
            pl.BlockSpec((BLK, H), lambda i: (i, 0)),
            pl.BlockSpec((BLK, 1), lambda i: (i, 0)),
            pl.BlockSpec((1, H), lambda i: (0, 0)),
        ],
        out_specs=[
            pl.BlockSpec((BLK, H), lambda i: (i, 0)),
            pl.BlockSpec((2, H), lambda i: (0, 0)),
        ],
        out_shape=[
            jax.ShapeDtypeStruct((N, H), jnp.float32),
            jax.ShapeDtypeStruct((2, H), jnp.float32),
        ],
    )(p, u, dinv, b.reshape(1, H))


def _norm_body(agg_ref, st_ref, g_ref, bt_ref, wn_ref, dinv_ref, u_ref):
    st = st_ref[...]
    m = st[0:1] * (1.0 / N)
    var = st[1:2] * (1.0 / N) - m * m
    rstd = lax.rsqrt(var + 1e-5)
    a = (agg_ref[...] - m) * (rstd * g_ref[...]) + bt_ref[...]
    a = jnp.where(a > 0.0, a, jnp.exp(a) - 1.0)
    u_ref[...] = dinv_ref[...] * jnp.dot(a, wn_ref[...],
                                         preferred_element_type=jnp.float32)


def _tc_norm(agg, st, g, bt, wn, dinv):
    return pl.pallas_call(
        _norm_body,
        grid=(NB,),
        in_specs=[
            pl.BlockSpec((BLK, H), lambda i: (i, 0)),
            pl.BlockSpec((2, H), lambda i: (0, 0)),
            pl.BlockSpec((1, H), lambda i: (0, 0)),
            pl.BlockSpec((1, H), lambda i: (0, 0)),
            pl.BlockSpec((H, H), lambda i: (0, 0)),
            pl.BlockSpec((BLK, 1), lambda i: (i, 0)),
        ],
        out_specs=pl.BlockSpec((BLK, H), lambda i: (i, 0)),
        out_shape=jax.ShapeDtypeStruct((N, H), jnp.float32),
    )(agg, st, g.reshape(1, H), bt.reshape(1, H), wn, dinv)


def _pool_body(agg_ref, st_ref, g_ref, bt_ref, batch_ref, ps_ref, pc_ref):
    i = pl.program_id(0)
    st = st_ref[...]
    m = st[0:1] * (1.0 / N)
    var = st[1:2] * (1.0 / N) - m * m
    rstd = lax.rsqrt(var + 1e-5)
    a = (agg_ref[...] - m) * (rstd * g_ref[...]) + bt_ref[...]
    a = jnp.maximum(a, 0.0)

    @pl.when(i == 0)
    def _():
        ps_ref[...] = jnp.zeros_like(ps_ref)
        pc_ref[...] = jnp.zeros_like(pc_ref)

    ids = batch_ref[...]
    iot = lax.broadcasted_iota(jnp.int32, (BLK, G), 1)
    oh = jnp.where(iot == ids, 1.0, 0.0)
    dn = (((0,), (0,)), ((), ()))
    ps_ref[...] += lax.dot_general(oh, a, dn,
                                   preferred_element_type=jnp.float32)
    pc_ref[...] += lax.dot_general(oh, jnp.ones((BLK, 1), jnp.float32), dn,
                                   preferred_element_type=jnp.float32)


def _tc_pool(agg, st, g, bt, batch2d):
    return pl.pallas_call(
        _pool_body,
        grid=(NB,),
        in_specs=[
            pl.BlockSpec((BLK, H), lambda i: (i, 0)),
            pl.BlockSpec((2, H), lambda i: (0, 0)),
            pl.BlockSpec((1, H), lambda i: (0, 0)),
            pl.BlockSpec((1, H), lambda i: (0, 0)),
            pl.BlockSpec((BLK, 1), lambda i: (i, 0)),
        ],
        out_specs=[
            pl.BlockSpec((G, H), lambda i: (0, 0)),
            pl.BlockSpec((G, 1), lambda i: (0, 0)),
        ],
        out_shape=[
            jax.ShapeDtypeStruct((G, H), jnp.float32),
            jax.ShapeDtypeStruct((G, 1), jnp.float32),
        ],
    )(agg, st, g.reshape(1, H), bt.reshape(1, H), batch2d)


def _final_body(ps_ref, pc_ref, wp1_ref, bp1_ref, wp2_ref, bp2_ref, out_ref):
    pooled = ps_ref[...] / jnp.maximum(pc_ref[...], 1.0)
    h1 = jnp.dot(pooled, wp1_ref[...],
                 preferred_element_type=jnp.float32) + bp1_ref[...]
    h1 = jnp.where(h1 > 0.0, h1, jnp.exp(h1) - 1.0)
    out_ref[...] = jnp.dot(h1, wp2_ref[...],
                           preferred_element_type=jnp.float32) + bp2_ref[...]


def _tc_final(ps, pc, Wp1, bp1, Wp2, bp2):
    return pl.pallas_call(
        _final_body,
        out_shape=jax.ShapeDtypeStruct((G, 1), jnp.float32),
    )(ps, pc, Wp1, bp1.reshape(1, H // 2), Wp2, bp2.reshape(1, 1))


def kernel(x, edge_index, batch, W_a1, b_a1, W_a2,
           Wg0, bg0, gamma0, beta0, Wg1, bg1, gamma1, beta1,
           Wg2, bg2, gamma2, beta2, Wg3, bg3, gamma3, beta3,
           Wp1, bp1, Wp2, bp2):
    pad = E_PAD - E
    pad_ar = jnp.arange(pad, dtype=jnp.int32)
    src_pad = jnp.concatenate(
        [edge_index[0], pad_ar % N]).reshape(CH_TOT, CHUNK)
    dst_pad = jnp.concatenate(
        [edge_index[1], N + pad_ar % (ACC_N - N)]).reshape(CH_TOT, CHUNK)
    batch2d = batch.reshape(N, 1)

    tmin = _tc_tmin(x)
    sums, cnts = _tc_seg(x, tmin)
    subst = _tc_attn(sums, cnts, W_a1, b_a1, W_a2)

    degp = _sc_deg(dst_pad)
    degpT = degp.T

    W0a = Wg0[:D]
    w0b = Wg0[D:D + 1]
    u, dinv = _tc_k0(x, degpT, subst, W0a, w0b, tmin)

    layers = [(bg0, gamma0, beta0, Wg1), (bg1, gamma1, beta1, Wg2),
              (bg2, gamma2, beta2, Wg3)]
    for b, g, bt, wn in layers:
        p = _sc_mp(src_pad, dst_pad, u)
        agg, st = _tc_comb(p, u, dinv, b)
        u = _tc_norm(agg, st, g, bt, wn, dinv)

    p = _sc_mp(src_pad, dst_pad, u)
    agg, st = _tc_comb(p, u, dinv, bg3)
    ps, pc = _tc_pool(agg, st, gamma3, beta3, batch2d)
    return _tc_final(ps, pc, Wp1, bp1, Wp2, bp2)


# speedup vs baseline: 3.3347x; 1.0158x over previous
import functools

import jax
import jax.numpy as jnp
from jax import lax
from jax.experimental import pallas as pl
from jax.experimental.pallas import tpu as pltpu
from jax.experimental.pallas import tpu_sc as plsc

N, E, D, H, G = 10000, 320000, 128, 128, 512
NSEG = 64
NC, NS = 2, 16
CHUNK = 128
NCH = 80
NCHD = 80
GRP = 16
CH_TOT = NC * NS * NCH
E_PAD = CH_TOT * CHUNK
ROWS_PT = 640
ACC_N = NS * ROWS_PT
BLK = 1000
NB = N // BLK


def _sc_deg_body(dst_hbm, out_hbm, idx_v, ones_v, zeros_v, acc):
    c = lax.axis_index("c")
    s = lax.axis_index("s")
    for k in range(CHUNK // 16):
        ones_v[pl.ds(k * 16, 16)] = jnp.ones((16,), jnp.float32)
    for k in range(ROWS_PT // 16):
        zeros_v[pl.ds(k * 16, 16)] = jnp.zeros((16,), jnp.float32)
    pltpu.sync_copy(zeros_v, acc.at[pl.ds(s * ROWS_PT, ROWS_PT)])
    base = pl.multiple_of((c * NS + s) * NCHD, 8)
    pltpu.sync_copy(dst_hbm.at[pl.ds(base, NCHD)], idx_v)
    plsc.subcore_barrier()

    def body(j, carry):
        pltpu.sync_copy(ones_v, acc.at[idx_v.at[j]], add=True)
        return carry

    lax.fori_loop(0, NCHD, body, 0)
    plsc.subcore_barrier()

    @pl.when(s == 0)
    def _():
        pltpu.sync_copy(acc, out_hbm.at[c])


def _sc_deg(dst_pad):
    mesh = plsc.VectorSubcoreMesh(core_axis_name="c", subcore_axis_name="s", num_cores=NC, num_subcores=NS)
    f = pl.kernel(
        _sc_deg_body,
        out_type=jax.ShapeDtypeStruct((NC, ACC_N), jnp.float32),
        mesh=mesh,
        scratch_types=[
            pltpu.VMEM((NCHD, CHUNK), jnp.int32),
            pltpu.VMEM((CHUNK,), jnp.float32),
            pltpu.VMEM((ROWS_PT,), jnp.float32),
            pltpu.VMEM_SHARED((ACC_N,), jnp.float32),
        ],
    )
    return f(dst_pad)


def _sc_mp_body(src_hbm, dst_hbm, u_hbm, out_hbm, srcv, dstv,
                rows0, rows1, sem0, sem1, acc):
    c = lax.axis_index("c")
    s = lax.axis_index("s")

    def zrow(i, carry):
        for k in range(H // 16):
            rows0[i, pl.ds(k * 16, 16)] = jnp.zeros((16,), jnp.float32)
        return carry

    lax.fori_loop(0, CHUNK, zrow, 0)
    for k in range(ROWS_PT // CHUNK):
        pltpu.sync_copy(rows0, acc.at[pl.ds(s * ROWS_PT + k * CHUNK, CHUNK)])

    plsc.subcore_barrier()

    base = (c * NS + s) * NCH

    def group(g, carry):
        gofs = pl.multiple_of(base + g * GRP, 8)
        pltpu.sync_copy(src_hbm.at[pl.ds(gofs, GRP)], srcv)
        pltpu.sync_copy(dst_hbm.at[pl.ds(gofs, GRP)], dstv)
        pltpu.async_copy(u_hbm.at[srcv.at[0]], rows0, sem0)

        def body(jj, carry2):
            j = 2 * jj
            pltpu.make_async_copy(u_hbm.at[srcv.at[j]], rows0, sem0).wait()
            pltpu.async_copy(u_hbm.at[srcv.at[j + 1]], rows1, sem1)
            pltpu.sync_copy(rows0, acc.at[dstv.at[j]], add=True)
            pltpu.make_async_copy(u_hbm.at[srcv.at[j + 1]], rows1, sem1).wait()

            @pl.when(jj < GRP // 2 - 1)
            def _():
                pltpu.async_copy(u_hbm.at[srcv.at[j + 2]], rows0, sem0)

            pltpu.sync_copy(rows1, acc.at[dstv.at[j + 1]], add=True)
            return carry2

        lax.fori_loop(0, GRP // 2, body, 0)
        return carry

    lax.fori_loop(0, NCH // GRP, group, 0)
    plsc.subcore_barrier()
    pltpu.sync_copy(acc.at[pl.ds(s * ROWS_PT, ROWS_PT)],
                    out_hbm.at[c, pl.ds(s * ROWS_PT, ROWS_PT)])


def _sc_mp(src_pad, dst_pad, u):
    mesh = plsc.VectorSubcoreMesh(core_axis_name="c", subcore_axis_name="s", num_cores=NC, num_subcores=NS)
    f = pl.kernel(
        _sc_mp_body,
        out_type=jax.ShapeDtypeStruct((NC, ACC_N, H), jnp.float32),
        mesh=mesh,
        scratch_types=[
            pltpu.VMEM((GRP, CHUNK), jnp.int32),
            pltpu.VMEM((GRP, CHUNK), jnp.int32),
            pltpu.VMEM((CHUNK, H), jnp.float32),
            pltpu.VMEM((CHUNK, H), jnp.float32),
            pltpu.SemaphoreType.DMA,
            pltpu.SemaphoreType.DMA,
            pltpu.VMEM_SHARED((ACC_N, H), jnp.float32),
        ],
    )
    return f(src_pad, dst_pad, u)


def _tmin_body(x_ref, out_ref, m_ref):
    i = pl.program_id(0)

    @pl.when(i == 0)
    def _():
        m_ref[0] = jnp.int32(2147483647)

    t = x_ref[...].astype(jnp.int32)
    m_ref[0] = jnp.minimum(m_ref[0], jnp.min(t[:, 5:6]))

    @pl.when(i == NB - 1)
    def _():
        out_ref[0, 0] = m_ref[0]


def _tc_tmin(x):
    return pl.pallas_call(
        _tmin_body,
        grid=(NB,),
        in_specs=[pl.BlockSpec((BLK, D), lambda i: (i, 0))],
        out_specs=pl.BlockSpec(memory_space=pltpu.SMEM),
        out_shape=jax.ShapeDtypeStruct((1, 1), jnp.int32),
        scratch_shapes=[pltpu.SMEM((1,), jnp.int32)],
    )(x)


def _seg_body(tmin_ref, x_ref, sums_ref, cnts_ref):
    i = pl.program_id(0)

    @pl.when(i == 0)
    def _():
        sums_ref[...] = jnp.zeros_like(sums_ref)
        cnts_ref[...] = jnp.zeros_like(cnts_ref)

    xb = x_ref[...]
    ids = xb[:, 5:6].astype(jnp.int32) - tmin_ref[0, 0]
    iot = lax.broadcasted_iota(jnp.int32, (BLK, NSEG), 1)
    oh = jnp.where((iot == ids) & (ids < NSEG), 1.0, 0.0)
    dn = (((0,), (0,)), ((), ()))
    sums_ref[...] += lax.dot_general(oh, xb, dn,
                                     preferred_element_type=jnp.float32)
    cnts_ref[...] += lax.dot_general(oh, jnp.ones((BLK, 1), jnp.float32), dn,
                                     preferred_element_type=jnp.float32)


def _tc_seg(x, tmin):
    return pl.pallas_call(
        _seg_body,
        grid=(NB,),
        in_specs=[
            pl.BlockSpec(memory_space=pltpu.SMEM),
            pl.BlockSpec((BLK, D), lambda i: (i, 0)),
        ],
        out_specs=[
            pl.BlockSpec((NSEG, D), lambda i: (0, 0)),
            pl.BlockSpec((NSEG, 1), lambda i: (0, 0)),
        ],
        out_shape=[
            jax.ShapeDtypeStruct((NSEG, D), jnp.float32),
            jax.ShapeDtypeStruct((NSEG, 1), jnp.float32),
        ],
    )(tmin, x)


def _attn_body(sums_ref, cnts_ref, wa1_ref, ba1_ref, wa2_ref, out_ref):
    cnts = cnts_ref[...]
    means = sums_ref[...] / jnp.maximum(cnts, 1.0)
    hmid = jnp.tanh(jnp.dot(means, wa1_ref[...],
                            preferred_element_type=jnp.float32) + ba1_ref[...])
    scores = jnp.dot(hmid, wa2_ref[...],
                     preferred_element_type=jnp.float32)
    scores = jnp.where(cnts > 0.0, scores, -1e30)
    mx = jnp.max(scores, axis=0, keepdims=True)
    e = jnp.exp(scores - mx)
    out_ref[...] = e / jnp.sum(e, axis=0, keepdims=True)


def _tc_attn(sums, cnts, W_a1, b_a1, W_a2):
    return pl.pallas_call(
        _attn_body,
        out_shape=jax.ShapeDtypeStruct((NSEG, 1), jnp.float32),
    )(sums, cnts, W_a1, b_a1.reshape(1, 64), W_a2)


def _k0_body(tmin_ref, x_ref, degp_ref, subst_ref, w0a_ref, w0b_ref,
             u_ref, dinv_ref):
    xb = x_ref[...]
    deg = degp_ref[:, 0:1] + degp_ref[:, 1:2] + 1.0
    dinv = lax.rsqrt(deg)
    ids = xb[:, 5:6].astype(jnp.int32) - tmin_ref[0, 0]
    ids = jnp.minimum(ids, NSEG - 1)
    iot = lax.broadcasted_iota(jnp.int32, (BLK, NSEG), 1)
    oh = jnp.where(iot == ids, 1.0, 0.0)
    attn = jnp.dot(oh, subst_ref[...],
                   preferred_element_type=jnp.float32)
    u = jnp.dot(xb, w0a_ref[...], preferred_element_type=jnp.float32)
    u = dinv * (u + attn * w0b_ref[...])
    u_ref[...] = u
    dinv_ref[...] = dinv


def _tc_k0(x, degpT, subst, W0a, w0b, tmin):
    return pl.pallas_call(
        _k0_body,
        grid=(NB,),
        in_specs=[
            pl.BlockSpec(memory_space=pltpu.SMEM),
            pl.BlockSpec((BLK, D), lambda i: (i, 0)),
            pl.BlockSpec((BLK, NC), lambda i: (i, 0)),
            pl.BlockSpec((NSEG, 1), lambda i: (0, 0)),
            pl.BlockSpec((D, H), lambda i: (0, 0)),
            pl.BlockSpec((1, H), lambda i: (0, 0)),
        ],
        out_specs=[
            pl.BlockSpec((BLK, H), lambda i: (i, 0)),
            pl.BlockSpec((BLK, 1), lambda i: (i, 0)),
        ],
        out_shape=[
            jax.ShapeDtypeStruct((N, H), jnp.float32),
            jax.ShapeDtypeStruct((N, 1), jnp.float32),
        ],
    )(tmin, x, degpT, subst, W0a, w0b)


def _layer_body(p_ref, u_ref, dinv_ref, b_ref, g_ref, bt_ref, wn_ref,
                un_ref, agg_s, st_s):
    ph = pl.program_id(0)
    i = pl.program_id(1)

    @pl.when((ph == 0) & (i == 0))
    def _():
        st_s[...] = jnp.zeros_like(st_s)

    @pl.when(ph == 0)
    def _():
        agg = dinv_ref[...] * (p_ref[0] + p_ref[1] + u_ref[...]) + b_ref[...]
        agg_s[pl.ds(i * BLK, BLK), :] = agg
        st_s[0:1, :] += jnp.sum(agg, axis=0, keepdims=True)
        st_s[1:2, :] += jnp.sum(agg * agg, axis=0, keepdims=True)

    @pl.when(ph == 1)
    def _():
        st = st_s[...]
        m = st[0:1, :] * (1.0 / N)
        var = st[1:2, :] * (1.0 / N) - m * m
        rstd = lax.rsqrt(var + 1e-5)
        a = (agg_s[pl.ds(i * BLK, BLK), :] - m) * (rstd * g_ref[...]) \
            + bt_ref[...]
        a = jnp.where(a > 0.0, a, jnp.exp(a) - 1.0)
        un_ref[...] = dinv_ref[...] * jnp.dot(
            a, wn_ref[...], preferred_element_type=jnp.float32)


def _tc_layer(p, u, dinv, b, g, bt, wn):
    return pl.pallas_call(
        _layer_body,
        grid=(2, NB),
        in_specs=[
            pl.BlockSpec((NC, BLK, H),
                         lambda ph, i: (0, jnp.where(ph == 0, i, 0), 0)),
            pl.BlockSpec((BLK, H),
                         lambda ph, i: (jnp.where(ph == 0, i, 0), 0)),
            pl.BlockSpec((BLK, 1), lambda ph, i: (i, 0)),
            pl.BlockSpec((1, H), lambda ph, i: (0, 0)),
            pl.BlockSpec((1, H), lambda ph, i: (0, 0)),
            pl.BlockSpec((1, H), lambda ph, i: (0, 0)),
            pl.BlockSpec((H, H), lambda ph, i: (0, 0)),
        ],
        out_specs=pl.BlockSpec((BLK, H),
                               lambda ph, i: (jnp.where(ph == 1, i, 0), 0)),
        out_shape=jax.ShapeDtypeStruct((N, H), jnp.float32),
        scratch_shapes=[
            pltpu.VMEM((N, H), jnp.float32),
            pltpu.VMEM((2, H), jnp.float32),
        ],
    )(p, u, dinv, b.reshape(1, H), g.reshape(1, H), bt.reshape(1, H), wn)


def _lpool_body(p_ref, u_ref, dinv_ref, b_ref, g_ref, bt_ref, batch_ref,
                ps_ref, pc_ref, agg_s, st_s):
    ph = pl.program_id(0)
    i = pl.program_id(1)

    @pl.when((ph == 0) & (i == 0))
    def _():
        st_s[...] = jnp.zeros_like(st_s)

    @pl.when(ph == 0)
    def _():
        agg = dinv_ref[...] * (p_ref[0] + p_ref[1] + u_ref[...]) + b_ref[...]
        agg_s[pl.ds(i * BLK, BLK), :] = agg
        st_s[0:1, :] += jnp.sum(agg, axis=0, keepdims=True)
        st_s[1:2, :] += jnp.sum(agg * agg, axis=0, keepdims=True)

    @pl.when((ph == 1) & (i == 0))
    def _():
        ps_ref[...] = jnp.zeros_like(ps_ref)
        pc_ref[...] = jnp.zeros_like(pc_ref)

    @pl.when(ph == 1)
    def _():
        st = st_s[...]
        m = st[0:1, :] * (1.0 / N)
        var = st[1:2, :] * (1.0 / N) - m * m
        rstd = lax.rsqrt(var + 1e-5)
        a = (agg_s[pl.ds(i * BLK, BLK), :] - m) * (rstd * g_ref[...]) \
            + bt_ref[...]
        a = jnp.maximum(a, 0.0)
        ids = batch_ref[...]
        iot = lax.broadcasted_iota(jnp.int32, (BLK, G), 1)
        oh = jnp.where(iot == ids, 1.0, 0.0)
        dn = (((0,), (0,)), ((), ()))
        ps_ref[...] += lax.dot_general(oh, a, dn,
                                       preferred_element_type=jnp.float32)
        pc_ref[...] += lax.dot_general(
            oh, jnp.ones((BLK, 1), jnp.float32), dn,
            preferred_element_type=jnp.float32)


def _tc_lpool(p, u, dinv, b, g, bt, batch2d):
    return pl.pallas_call(
        _lpool_body,
        grid=(2, NB),
        in_specs=[
            pl.BlockSpec((NC, BLK, H),
                         lambda ph, i: (0, jnp.where(ph == 0, i, 0), 0)),
            pl.BlockSpec((BLK, H),
                         lambda ph, i: (jnp.where(ph == 0, i, 0), 0)),
            pl.BlockSpec((BLK, 1), lambda ph, i: (i, 0)),
            pl.BlockSpec((1, H), lambda ph, i: (0, 0)),
            pl.BlockSpec((1, H), lambda ph, i: (0, 0)),
            pl.BlockSpec((1, H), lambda ph, i: (0, 0)),
            pl.BlockSpec((BLK, 1),
                         lambda ph, i: (jnp.where(ph == 1, i, 0), 0)),
        ],
        out_specs=[
            pl.BlockSpec((G, H), lambda ph, i: (0, 0)),
            pl.BlockSpec((G, 1), lambda ph, i: (0, 0)),
        ],
        out_shape=[
            jax.ShapeDtypeStruct((G, H), jnp.float32),
            jax.ShapeDtypeStruct((G, 1), jnp.float32),
        ],
        scratch_shapes=[
            pltpu.VMEM((N, H), jnp.float32),
            pltpu.VMEM((2, H), jnp.float32),
        ],
    )(p, u, dinv, b.reshape(1, H), g.reshape(1, H), bt.reshape(1, H),
      batch2d)


def _final_body(ps_ref, pc_ref, wp1_ref, bp1_ref, wp2_ref, bp2_ref, out_ref):
    pooled = ps_ref[...] / jnp.maximum(pc_ref[...], 1.0)
    h1 = jnp.dot(pooled, wp1_ref[...],
                 preferred_element_type=jnp.float32) + bp1_ref[...]
    h1 = jnp.where(h1 > 0.0, h1, jnp.exp(h1) - 1.0)
    out_ref[...] = jnp.dot(h1, wp2_ref[...],
                           preferred_element_type=jnp.float32) + bp2_ref[...]


def _tc_final(ps, pc, Wp1, bp1, Wp2, bp2):
    return pl.pallas_call(
        _final_body,
        out_shape=jax.ShapeDtypeStruct((G, 1), jnp.float32),
    )(ps, pc, Wp1, bp1.reshape(1, H // 2), Wp2, bp2.reshape(1, 1))


def kernel(x, edge_index, batch, W_a1, b_a1, W_a2,
           Wg0, bg0, gamma0, beta0, Wg1, bg1, gamma1, beta1,
           Wg2, bg2, gamma2, beta2, Wg3, bg3, gamma3, beta3,
           Wp1, bp1, Wp2, bp2):
    pad = E_PAD - E
    pad_ar = jnp.arange(pad, dtype=jnp.int32)
    src_pad = jnp.concatenate(
        [edge_index[0], pad_ar % N]).reshape(CH_TOT, CHUNK)
    dst_pad = jnp.concatenate(
        [edge_index[1], N + pad_ar % (ACC_N - N)]).reshape(CH_TOT, CHUNK)
    batch2d = batch.reshape(N, 1)

    tmin = _tc_tmin(x)
    sums, cnts = _tc_seg(x, tmin)
    subst = _tc_attn(sums, cnts, W_a1, b_a1, W_a2)

    degp = _sc_deg(dst_pad)
    degpT = degp.T

    W0a = Wg0[:D]
    w0b = Wg0[D:D + 1]
    u, dinv = _tc_k0(x, degpT, subst, W0a, w0b, tmin)

    layers = [(bg0, gamma0, beta0, Wg1), (bg1, gamma1, beta1, Wg2),
              (bg2, gamma2, beta2, Wg3)]
    for b, g, bt, wn in layers:
        p = _sc_mp(src_pad, dst_pad, u)
        u = _tc_layer(p, u, dinv, b, g, bt, wn)

    p = _sc_mp(src_pad, dst_pad, u)
    ps, pc = _tc_lpool(p, u, dinv, bg3, gamma3, beta3, batch2d)
    return _tc_final(ps, pc, Wp1, bp1, Wp2, bp2)


# speedup vs baseline: 3.4522x; 1.0352x over previous
import functools

import jax
import jax.numpy as jnp
from jax import lax
from jax.experimental import pallas as pl
from jax.experimental.pallas import tpu as pltpu
from jax.experimental.pallas import tpu_sc as plsc

N, E, D, H, G = 10000, 320000, 128, 128, 512
NSEG = 64
NC, NS = 2, 16
CHUNK = 128
NCH = 80
NCHD = 80
GRP = 40
CH_TOT = NC * NS * NCH
E_PAD = CH_TOT * CHUNK
ROWS_PT = 640
ACC_N = NS * ROWS_PT
BLK = 1000
NB = N // BLK


def _sc_deg_body(dst_hbm, out_hbm, idx_v, ones_v, zeros_v, acc):
    c = lax.axis_index("c")
    s = lax.axis_index("s")
    for k in range(CHUNK // 16):
        ones_v[pl.ds(k * 16, 16)] = jnp.ones((16,), jnp.float32)
    for k in range(ROWS_PT // 16):
        zeros_v[pl.ds(k * 16, 16)] = jnp.zeros((16,), jnp.float32)
    pltpu.sync_copy(zeros_v, acc.at[pl.ds(s * ROWS_PT, ROWS_PT)])
    base = pl.multiple_of((c * NS + s) * NCHD, 8)
    pltpu.sync_copy(dst_hbm.at[pl.ds(base, NCHD)], idx_v)
    plsc.subcore_barrier()

    def body(j, carry):
        pltpu.sync_copy(ones_v, acc.at[idx_v.at[j]], add=True)
        return carry

    lax.fori_loop(0, NCHD, body, 0)
    plsc.subcore_barrier()

    @pl.when(s == 0)
    def _():
        pltpu.sync_copy(acc, out_hbm.at[c])


def _sc_deg(dst_pad):
    mesh = plsc.VectorSubcoreMesh(core_axis_name="c", subcore_axis_name="s", num_cores=NC, num_subcores=NS)
    f = pl.kernel(
        _sc_deg_body,
        out_type=jax.ShapeDtypeStruct((NC, ACC_N), jnp.float32),
        mesh=mesh,
        scratch_types=[
            pltpu.VMEM((NCHD, CHUNK), jnp.int32),
            pltpu.VMEM((CHUNK,), jnp.float32),
            pltpu.VMEM((ROWS_PT,), jnp.float32),
            pltpu.VMEM_SHARED((ACC_N,), jnp.float32),
        ],
    )
    return f(dst_pad)


def _sc_mp_body(src_hbm, dst_hbm, u_hbm, out_hbm, srcv, dstv,
                rows0, rows1, sem0, sem1, sem2, sem3, acc):
    c = lax.axis_index("c")
    s = lax.axis_index("s")

    def zrow(i, carry):
        for k in range(H // 16):
            rows0[i, pl.ds(k * 16, 16)] = jnp.zeros((16,), jnp.float32)
        return carry

    lax.fori_loop(0, CHUNK, zrow, 0)
    for k in range(ROWS_PT // CHUNK):
        pltpu.sync_copy(rows0, acc.at[pl.ds(s * ROWS_PT + k * CHUNK, CHUNK)])

    plsc.subcore_barrier()

    base = (c * NS + s) * NCH

    def gath(j, rows, sem):
        return pltpu.async_copy(u_hbm.at[srcv.at[j]], rows, sem)

    def gath_wait(j, rows, sem):
        pltpu.make_async_copy(u_hbm.at[srcv.at[j]], rows, sem).wait()

    def scat(j, rows, sem):
        return pltpu.async_copy(rows, acc.at[dstv.at[j]], sem, add=True)

    def scat_wait(j, rows, sem):
        pltpu.make_async_copy(rows, acc.at[dstv.at[j]], sem).wait()

    def group(g, carry):
        gofs = pl.multiple_of(g * GRP, 8)
        pltpu.sync_copy(src_hbm.at[pl.ds(base + gofs, GRP)], srcv)
        pltpu.sync_copy(dst_hbm.at[pl.ds(base + gofs, GRP)], dstv)
        gath(0, rows0, sem0)
        gath_wait(0, rows0, sem0)
        gath(1, rows1, sem1)
        scat(0, rows0, sem2)

        def body(jj, carry2):
            j = 2 * jj
            gath_wait(j + 1, rows1, sem1)
            scat_wait(j, rows0, sem2)
            gath(j + 2, rows0, sem0)
            scat(j + 1, rows1, sem3)
            gath_wait(j + 2, rows0, sem0)
            scat_wait(j + 1, rows1, sem3)
            gath(j + 3, rows1, sem1)
            scat(j + 2, rows0, sem2)
            return carry2

        lax.fori_loop(0, GRP // 2 - 1, body, 0)
        gath_wait(GRP - 1, rows1, sem1)
        scat_wait(GRP - 2, rows0, sem2)
        scat(GRP - 1, rows1, sem3)
        scat_wait(GRP - 1, rows1, sem3)
        return carry

    lax.fori_loop(0, NCH // GRP, group, 0)
    plsc.subcore_barrier()
    pltpu.sync_copy(acc.at[pl.ds(s * ROWS_PT, ROWS_PT)],
                    out_hbm.at[c, pl.ds(s * ROWS_PT, ROWS_PT)])


def _sc_mp(src_pad, dst_pad, u):
    mesh = plsc.VectorSubcoreMesh(core_axis_name="c", subcore_axis_name="s", num_cores=NC, num_subcores=NS)
    f = pl.kernel(
        _sc_mp_body,
        out_type=jax.ShapeDtypeStruct((NC, ACC_N, H), jnp.float32),
        mesh=mesh,
        scratch_types=[
            pltpu.VMEM((GRP, CHUNK), jnp.int32),
            pltpu.VMEM((GRP, CHUNK), jnp.int32),
            pltpu.VMEM((CHUNK, H), jnp.float32),
            pltpu.VMEM((CHUNK, H), jnp.float32),
            pltpu.SemaphoreType.DMA,
            pltpu.SemaphoreType.DMA,
            pltpu.SemaphoreType.DMA,
            pltpu.SemaphoreType.DMA,
            pltpu.VMEM_SHARED((ACC_N, H), jnp.float32),
        ],
    )
    return f(src_pad, dst_pad, u)


def _tmin_body(x_ref, out_ref, m_ref):
    i = pl.program_id(0)

    @pl.when(i == 0)
    def _():
        m_ref[0] = jnp.int32(2147483647)

    t = x_ref[...].astype(jnp.int32)
    m_ref[0] = jnp.minimum(m_ref[0], jnp.min(t[:, 5:6]))

    @pl.when(i == NB - 1)
    def _():
        out_ref[0, 0] = m_ref[0]


def _tc_tmin(x):
    return pl.pallas_call(
        _tmin_body,
        grid=(NB,),
        in_specs=[pl.BlockSpec((BLK, D), lambda i: (i, 0))],
        out_specs=pl.BlockSpec(memory_space=pltpu.SMEM),
        out_shape=jax.ShapeDtypeStruct((1, 1), jnp.int32),
        scratch_shapes=[pltpu.SMEM((1,), jnp.int32)],
    )(x)


def _seg_body(tmin_ref, x_ref, sums_ref, cnts_ref):
    i = pl.program_id(0)

    @pl.when(i == 0)
    def _():
        sums_ref[...] = jnp.zeros_like(sums_ref)
        cnts_ref[...] = jnp.zeros_like(cnts_ref)

    xb = x_ref[...]
    ids = xb[:, 5:6].astype(jnp.int32) - tmin_ref[0, 0]
    iot = lax.broadcasted_iota(jnp.int32, (BLK, NSEG), 1)
    oh = jnp.where((iot == ids) & (ids < NSEG), 1.0, 0.0)
    dn = (((0,), (0,)), ((), ()))
    sums_ref[...] += lax.dot_general(oh, xb, dn,
                                     preferred_element_type=jnp.float32)
    cnts_ref[...] += lax.dot_general(oh, jnp.ones((BLK, 1), jnp.float32), dn,
                                     preferred_element_type=jnp.float32)


def _tc_seg(x, tmin):
    return pl.pallas_call(
        _seg_body,
        grid=(NB,),
        in_specs=[
            pl.BlockSpec(memory_space=pltpu.SMEM),
            pl.BlockSpec((BLK, D), lambda i: (i, 0)),
        ],
        out_specs=[
            pl.BlockSpec((NSEG, D), lambda i: (0, 0)),
            pl.BlockSpec((NSEG, 1), lambda i: (0, 0)),
        ],
        out_shape=[
            jax.ShapeDtypeStruct((NSEG, D), jnp.float32),
            jax.ShapeDtypeStruct((NSEG, 1), jnp.float32),
        ],
    )(tmin, x)


def _attn_body(sums_ref, cnts_ref, wa1_ref, ba1_ref, wa2_ref, out_ref):
    cnts = cnts_ref[...]
    means = sums_ref[...] / jnp.maximum(cnts, 1.0)
    hmid = jnp.tanh(jnp.dot(means, wa1_ref[...],
                            preferred_element_type=jnp.float32) + ba1_ref[...])
    scores = jnp.dot(hmid, wa2_ref[...],
                     preferred_element_type=jnp.float32)
    scores = jnp.where(cnts > 0.0, scores, -1e30)
    mx = jnp.max(scores, axis=0, keepdims=True)
    e = jnp.exp(scores - mx)
    out_ref[...] = e / jnp.sum(e, axis=0, keepdims=True)


def _tc_attn(sums, cnts, W_a1, b_a1, W_a2):
    return pl.pallas_call(
        _attn_body,
        out_shape=jax.ShapeDtypeStruct((NSEG, 1), jnp.float32),
    )(sums, cnts, W_a1, b_a1.reshape(1, 64), W_a2)


def _k0_body(tmin_ref, x_ref, degp_ref, subst_ref, w0a_ref, w0b_ref,
             u_ref, dinv_ref):
    xb = x_ref[...]
    deg = degp_ref[:, 0:1] + degp_ref[:, 1:2] + 1.0
    dinv = lax.rsqrt(deg)
    ids = xb[:, 5:6].astype(jnp.int32) - tmin_ref[0, 0]
    ids = jnp.minimum(ids, NSEG - 1)
    iot = lax.broadcasted_iota(jnp.int32, (BLK, NSEG), 1)
    oh = jnp.where(iot == ids, 1.0, 0.0)
    attn = jnp.dot(oh, subst_ref[...],
                   preferred_element_type=jnp.float32)
    u = jnp.dot(xb, w0a_ref[...], preferred_element_type=jnp.float32)
    u = dinv * (u + attn * w0b_ref[...])
    u_ref[...] = u
    dinv_ref[...] = dinv


def _tc_k0(x, degpT, subst, W0a, w0b, tmin):
    return pl.pallas_call(
        _k0_body,
        grid=(NB,),
        in_specs=[
            pl.BlockSpec(memory_space=pltpu.SMEM),
            pl.BlockSpec((BLK, D), lambda i: (i, 0)),
            pl.BlockSpec((BLK, NC), lambda i: (i, 0)),
            pl.BlockSpec((NSEG, 1), lambda i: (0, 0)),
            pl.BlockSpec((D, H), lambda i: (0, 0)),
            pl.BlockSpec((1, H), lambda i: (0, 0)),
        ],
        out_specs=[
            pl.BlockSpec((BLK, H), lambda i: (i, 0)),
            pl.BlockSpec((BLK, 1), lambda i: (i, 0)),
        ],
        out_shape=[
            jax.ShapeDtypeStruct((N, H), jnp.float32),
            jax.ShapeDtypeStruct((N, 1), jnp.float32),
        ],
    )(tmin, x, degpT, subst, W0a, w0b)


def _layer_body(p_ref, u_ref, dinv_ref, b_ref, g_ref, bt_ref, wn_ref,
                un_ref, agg_s, st_s):
    ph = pl.program_id(0)
    i = pl.program_id(1)

    @pl.when((ph == 0) & (i == 0))
    def _():
        st_s[...] = jnp.zeros_like(st_s)

    @pl.when(ph == 0)
    def _():
        agg = dinv_ref[...] * (p_ref[0] + p_ref[1] + u_ref[...]) + b_ref[...]
        agg_s[pl.ds(i * BLK, BLK), :] = agg
        st_s[0:1, :] += jnp.sum(agg, axis=0, keepdims=True)
        st_s[1:2, :] += jnp.sum(agg * agg, axis=0, keepdims=True)

    @pl.when(ph == 1)
    def _():
        st = st_s[...]
        m = st[0:1, :] * (1.0 / N)
        var = st[1:2, :] * (1.0 / N) - m * m
        rstd = lax.rsqrt(var + 1e-5)
        a = (agg_s[pl.ds(i * BLK, BLK), :] - m) * (rstd * g_ref[...]) \
            + bt_ref[...]
        a = jnp.where(a > 0.0, a, jnp.exp(a) - 1.0)
        un_ref[...] = dinv_ref[...] * jnp.dot(
            a, wn_ref[...], preferred_element_type=jnp.float32)


def _tc_layer(p, u, dinv, b, g, bt, wn):
    return pl.pallas_call(
        _layer_body,
        grid=(2, NB),
        in_specs=[
            pl.BlockSpec((NC, BLK, H),
                         lambda ph, i: (0, jnp.where(ph == 0, i, 0), 0)),
            pl.BlockSpec((BLK, H),
                         lambda ph, i: (jnp.where(ph == 0, i, 0), 0)),
            pl.BlockSpec((BLK, 1), lambda ph, i: (i, 0)),
            pl.BlockSpec((1, H), lambda ph, i: (0, 0)),
            pl.BlockSpec((1, H), lambda ph, i: (0, 0)),
            pl.BlockSpec((1, H), lambda ph, i: (0, 0)),
            pl.BlockSpec((H, H), lambda ph, i: (0, 0)),
        ],
        out_specs=pl.BlockSpec((BLK, H),
                               lambda ph, i: (jnp.where(ph == 1, i, 0), 0)),
        out_shape=jax.ShapeDtypeStruct((N, H), jnp.float32),
        scratch_shapes=[
            pltpu.VMEM((N, H), jnp.float32),
            pltpu.VMEM((2, H), jnp.float32),
        ],
    )(p, u, dinv, b.reshape(1, H), g.reshape(1, H), bt.reshape(1, H), wn)


def _lpool_body(p_ref, u_ref, dinv_ref, b_ref, g_ref, bt_ref, batch_ref,
                ps_ref, pc_ref, agg_s, st_s):
    ph = pl.program_id(0)
    i = pl.program_id(1)

    @pl.when((ph == 0) & (i == 0))
    def _():
        st_s[...] = jnp.zeros_like(st_s)

    @pl.when(ph == 0)
    def _():
        agg = dinv_ref[...] * (p_ref[0] + p_ref[1] + u_ref[...]) + b_ref[...]
        agg_s[pl.ds(i * BLK, BLK), :] = agg
        st_s[0:1, :] += jnp.sum(agg, axis=0, keepdims=True)
        st_s[1:2, :] += jnp.sum(agg * agg, axis=0, keepdims=True)

    @pl.when((ph == 1) & (i == 0))
    def _():
        ps_ref[...] = jnp.zeros_like(ps_ref)
        pc_ref[...] = jnp.zeros_like(pc_ref)

    @pl.when(ph == 1)
    def _():
        st = st_s[...]
        m = st[0:1, :] * (1.0 / N)
        var = st[1:2, :] * (1.0 / N) - m * m
        rstd = lax.rsqrt(var + 1e-5)
        a = (agg_s[pl.ds(i * BLK, BLK), :] - m) * (rstd * g_ref[...]) \
            + bt_ref[...]
        a = jnp.maximum(a, 0.0)
        ids = batch_ref[...]
        iot = lax.broadcasted_iota(jnp.int32, (BLK, G), 1)
        oh = jnp.where(iot == ids, 1.0, 0.0)
        dn = (((0,), (0,)), ((), ()))
        ps_ref[...] += lax.dot_general(oh, a, dn,
                                       preferred_element_type=jnp.float32)
        pc_ref[...] += lax.dot_general(
            oh, jnp.ones((BLK, 1), jnp.float32), dn,
            preferred_element_type=jnp.float32)


def _tc_lpool(p, u, dinv, b, g, bt, batch2d):
    return pl.pallas_call(
        _lpool_body,
        grid=(2, NB),
        in_specs=[
            pl.BlockSpec((NC, BLK, H),
                         lambda ph, i: (0, jnp.where(ph == 0, i, 0), 0)),
            pl.BlockSpec((BLK, H),
                         lambda ph, i: (jnp.where(ph == 0, i, 0), 0)),
            pl.BlockSpec((BLK, 1), lambda ph, i: (i, 0)),
            pl.BlockSpec((1, H), lambda ph, i: (0, 0)),
            pl.BlockSpec((1, H), lambda ph, i: (0, 0)),
            pl.BlockSpec((1, H), lambda ph, i: (0, 0)),
            pl.BlockSpec((BLK, 1),
                         lambda ph, i: (jnp.where(ph == 1, i, 0), 0)),
        ],
        out_specs=[
            pl.BlockSpec((G, H), lambda ph, i: (0, 0)),
            pl.BlockSpec((G, 1), lambda ph, i: (0, 0)),
        ],
        out_shape=[
            jax.ShapeDtypeStruct((G, H), jnp.float32),
            jax.ShapeDtypeStruct((G, 1), jnp.float32),
        ],
        scratch_shapes=[
            pltpu.VMEM((N, H), jnp.float32),
            pltpu.VMEM((2, H), jnp.float32),
        ],
    )(p, u, dinv, b.reshape(1, H), g.reshape(1, H), bt.reshape(1, H),
      batch2d)


def _final_body(ps_ref, pc_ref, wp1_ref, bp1_ref, wp2_ref, bp2_ref, out_ref):
    pooled = ps_ref[...] / jnp.maximum(pc_ref[...], 1.0)
    h1 = jnp.dot(pooled, wp1_ref[...],
                 preferred_element_type=jnp.float32) + bp1_ref[...]
    h1 = jnp.where(h1 > 0.0, h1, jnp.exp(h1) - 1.0)
    out_ref[...] = jnp.dot(h1, wp2_ref[...],
                           preferred_element_type=jnp.float32) + bp2_ref[...]


def _tc_final(ps, pc, Wp1, bp1, Wp2, bp2):
    return pl.pallas_call(
        _final_body,
        out_shape=jax.ShapeDtypeStruct((G, 1), jnp.float32),
    )(ps, pc, Wp1, bp1.reshape(1, H // 2), Wp2, bp2.reshape(1, 1))


def kernel(x, edge_index, batch, W_a1, b_a1, W_a2,
           Wg0, bg0, gamma0, beta0, Wg1, bg1, gamma1, beta1,
           Wg2, bg2, gamma2, beta2, Wg3, bg3, gamma3, beta3,
           Wp1, bp1, Wp2, bp2):
    pad = E_PAD - E
    pad_ar = jnp.arange(pad, dtype=jnp.int32)
    src_pad = jnp.concatenate(
        [edge_index[0], pad_ar % N]).reshape(CH_TOT, CHUNK)
    dst_pad = jnp.concatenate(
        [edge_index[1], N + pad_ar % (ACC_N - N)]).reshape(CH_TOT, CHUNK)
    batch2d = batch.reshape(N, 1)

    tmin = _tc_tmin(x)
    sums, cnts = _tc_seg(x, tmin)
    subst = _tc_attn(sums, cnts, W_a1, b_a1, W_a2)

    degp = _sc_deg(dst_pad)
    degpT = degp.T

    W0a = Wg0[:D]
    w0b = Wg0[D:D + 1]
    u, dinv = _tc_k0(x, degpT, subst, W0a, w0b, tmin)

    layers = [(bg0, gamma0, beta0, Wg1), (bg1, gamma1, beta1, Wg2),
              (bg2, gamma2, beta2, Wg3)]
    for b, g, bt, wn in layers:
        p = _sc_mp(src_pad, dst_pad, u)
        u = _tc_layer(p, u, dinv, b, g, bt, wn)

    p = _sc_mp(src_pad, dst_pad, u)
    ps, pc = _tc_lpool(p, u, dinv, bg3, gamma3, beta3, batch2d)
    return _tc_final(ps, pc, Wp1, bp1, Wp2, bp2)
